# baseline (speedup 1.0000x reference)
import functools

import jax
import jax.numpy as jnp
from jax import lax
from jax.experimental import pallas as pl
from jax.experimental.pallas import tpu as pltpu
from jax.experimental.pallas import tpu_sc as plsc

BUCKETS = 1000000
ED = 64
MD = 128
NC, NS, LANES = 2, 16, 16
NW = NC * NS

CHUNK = 1024
IPS = 128
NSTR = CHUNK // IPS


def _sc_hash_gather(prev, curr, embed, total):
    b_per_w = total // NW
    n_chunks = b_per_w // CHUNK

    @functools.partial(
        pl.kernel,
        out_type=jax.ShapeDtypeStruct((total, ED), jnp.float32),
        mesh=plsc.VectorSubcoreMesh(core_axis_name="c", subcore_axis_name="s"),
        scratch_types=[
            pltpu.VMEM((CHUNK,), jnp.int32),
            pltpu.VMEM((CHUNK,), jnp.int32),
            pltpu.VMEM((NSTR, IPS), jnp.int32),
            pltpu.VMEM((CHUNK, ED), jnp.float32),
            pltpu.SemaphoreType.DMA,
        ],
        compiler_params=pltpu.CompilerParams(use_tc_tiling_on_sc=False),
    )
    def k(prev_hbm, curr_hbm, embed_hbm, out_hbm, prev_v, curr_v, idx_v, rows_v, sem):
        i32 = jnp.int32
        wid = lax.axis_index("s") * i32(NC) + lax.axis_index("c")
        base = wid * i32(b_per_w)

        @pl.loop(i32(0), i32(n_chunks))
        def chunk_body(ci):
            off = base + ci * i32(CHUNK)
            pltpu.sync_copy(prev_hbm.at[pl.ds(off, CHUNK)], prev_v)
            pltpu.sync_copy(curr_hbm.at[pl.ds(off, CHUNK)], curr_v)

            @pl.loop(i32(0), i32(NSTR))
            def hash_row(s):
                for g in range(IPS // LANES):
                    o = s * i32(IPS) + i32(g * LANES)
                    p = prev_v[pl.ds(o, LANES)]
                    c = curr_v[pl.ds(o, LANES)]
                    a = (p * i32(16000)) % i32(BUCKETS)
                    b = (p * i32(15337)) % i32(BUCKETS)
                    idx_v[s, pl.ds(g * LANES, LANES)] = (a + b + c) % i32(BUCKETS)

            descs = []
            for s in range(NSTR):
                descs.append(pltpu.async_copy(
                    embed_hbm.at[idx_v.at[i32(s)]],
                    rows_v.at[pl.ds(s * IPS, IPS), :],
                    sem))
            for d in descs:
                d.wait()
            pltpu.sync_copy(rows_v, out_hbm.at[pl.ds(off, CHUNK), :])

    return k(prev, curr, embed)


def _tc_project(e, W, total):
    MBLK = 4096

    def body(e_ref, w_ref, o_ref):
        o_ref[...] = lax.dot_general(
            e_ref[...], w_ref[...],
            (((1,), (1,)), ((), ())),
            preferred_element_type=jnp.float32)

    return pl.pallas_call(
        body,
        grid=(total // MBLK,),
        in_specs=[
            pl.BlockSpec((MBLK, ED), lambda i: (i, jnp.int32(0))),
            pl.BlockSpec((MD, ED), lambda i: (jnp.int32(0), jnp.int32(0))),
        ],
        out_specs=pl.BlockSpec((MBLK, MD), lambda i: (i, jnp.int32(0))),
        out_shape=jax.ShapeDtypeStruct((total, MD), jnp.float32),
    )(e, W)


def kernel(prev_tok, curr_tok, embed, W):
    B, L = prev_tok.shape
    total = B * L
    prev = prev_tok.astype(jnp.int32).reshape(total)
    curr = curr_tok.astype(jnp.int32).reshape(total)
    e = _sc_hash_gather(prev, curr, embed.astype(jnp.float32), total)
    out = _tc_project(e, W.astype(jnp.float32), total)
    return out.reshape(B, L, MD)


# speedup vs baseline: 1.2188x; 1.2188x over previous
import functools

import jax
import jax.numpy as jnp
from jax import lax
from jax.experimental import pallas as pl
from jax.experimental.pallas import tpu as pltpu
from jax.experimental.pallas import tpu_sc as plsc

BUCKETS = 1000000
ED = 64
MD = 128
NC, NS, LANES = 2, 16, 16
NW = NC * NS

CHUNK = 1024
IPS = 128
NSTR = CHUNK // IPS


def _sc_hash_gather(prev, curr, embed, total):
    half = total // 2
    b_per_w = total // NW
    n_chunks = b_per_w // CHUNK

    @functools.partial(
        pl.kernel,
        out_type=jax.ShapeDtypeStruct((half, 2 * ED), jnp.float32),
        mesh=plsc.VectorSubcoreMesh(core_axis_name="c", subcore_axis_name="s"),
        scratch_types=[
            pltpu.VMEM((CHUNK,), jnp.int32),
            pltpu.VMEM((CHUNK,), jnp.int32),
            pltpu.VMEM((NSTR, IPS), jnp.int32),
            pltpu.VMEM((CHUNK, ED), jnp.float32),
            pltpu.SemaphoreType.DMA,
        ],
        compiler_params=pltpu.CompilerParams(use_tc_tiling_on_sc=False),
    )
    def k(prev_hbm, curr_hbm, embed_hbm, out_hbm, prev_v, curr_v, idx_v, rows_v, sem):
        i32 = jnp.int32
        wid = lax.axis_index("s") * i32(NC) + lax.axis_index("c")
        base = wid * i32(b_per_w)

        @pl.loop(i32(0), i32(n_chunks))
        def chunk_body(ci):
            off = base + ci * i32(CHUNK)
            pltpu.sync_copy(prev_hbm.at[pl.ds(off, CHUNK)], prev_v)
            pltpu.sync_copy(curr_hbm.at[pl.ds(off, CHUNK)], curr_v)

            @pl.loop(i32(0), i32(NSTR))
            def hash_row(s):
                for g in range(IPS // LANES):
                    o = s * i32(IPS) + i32(g * LANES)
                    p = prev_v[pl.ds(o, LANES)]
                    c = curr_v[pl.ds(o, LANES)]
                    a = (p * i32(16000)) % i32(BUCKETS)
                    b = (p * i32(15337)) % i32(BUCKETS)
                    idx_v[s, pl.ds(g * LANES, LANES)] = (a + b + c) % i32(BUCKETS)

            descs = []
            for s in range(NSTR):
                descs.append(pltpu.async_copy(
                    embed_hbm.at[idx_v.at[i32(s)]],
                    rows_v.at[pl.ds(s * IPS, IPS), :],
                    sem))
            for d in descs:
                d.wait()

            @pl.when(wid < i32(NW // 2))
            def _():
                pltpu.sync_copy(
                    rows_v, out_hbm.at[pl.ds(off, CHUNK), pl.ds(i32(0), ED)])

            @pl.when(wid >= i32(NW // 2))
            def _():
                pltpu.sync_copy(
                    rows_v,
                    out_hbm.at[pl.ds(off - i32(half), CHUNK), pl.ds(i32(ED), ED)])

    return k(prev, curr, embed)


def _tc_project(e2, W, total):
    half = total // 2
    PBLK = 2048

    def body(e_ref, w_ref, o_ref):
        x = e_ref[...]
        w = w_ref[...]
        lo = x[:, :ED]
        hi = x[:, ED:]
        o_ref[0] = lax.dot_general(
            lo, w, (((1,), (1,)), ((), ())), preferred_element_type=jnp.float32)
        o_ref[1] = lax.dot_general(
            hi, w, (((1,), (1,)), ((), ())), preferred_element_type=jnp.float32)

    return pl.pallas_call(
        body,
        grid=(half // PBLK,),
        in_specs=[
            pl.BlockSpec((PBLK, 2 * ED), lambda i: (i, jnp.int32(0))),
            pl.BlockSpec((MD, ED), lambda i: (jnp.int32(0), jnp.int32(0))),
        ],
        out_specs=pl.BlockSpec(
            (2, PBLK, MD), lambda i: (jnp.int32(0), i, jnp.int32(0))),
        out_shape=jax.ShapeDtypeStruct((2, half, MD), jnp.float32),
    )(e2, W)


def kernel(prev_tok, curr_tok, embed, W):
    B, L = prev_tok.shape
    total = B * L
    prev = prev_tok.astype(jnp.int32).reshape(total)
    curr = curr_tok.astype(jnp.int32).reshape(total)
    e2 = _sc_hash_gather(prev, curr, embed.astype(jnp.float32), total)
    out = _tc_project(e2, W.astype(jnp.float32), total)
    return out.reshape(B, L, MD)


# speedup vs baseline: 1.2343x; 1.0127x over previous
import functools

import jax
import jax.numpy as jnp
from jax import lax
from jax.experimental import pallas as pl
from jax.experimental.pallas import tpu as pltpu
from jax.experimental.pallas import tpu_sc as plsc

BUCKETS = 1000000
ED = 64
MD = 128
NC, NS, LANES = 2, 16, 16
NW = NC * NS

CHUNK = 512
IPS = 128
NSTR = CHUNK // IPS
NBUF = 2


def _sc_hash_gather(prev, curr, embed, total):
    half = total // 2
    b_per_w = total // NW
    n_chunks = b_per_w // CHUNK
    n_groups = n_chunks // NBUF

    @functools.partial(
        pl.kernel,
        out_type=jax.ShapeDtypeStruct((half, 2 * ED), jnp.float32),
        mesh=plsc.VectorSubcoreMesh(core_axis_name="c", subcore_axis_name="s"),
        scratch_types=[
            pltpu.VMEM((b_per_w,), jnp.int32),
            pltpu.VMEM((b_per_w,), jnp.int32),
            [pltpu.VMEM((NSTR, IPS), jnp.int32) for _ in range(NBUF)],
            [pltpu.VMEM((CHUNK, ED), jnp.float32) for _ in range(NBUF)],
            [pltpu.SemaphoreType.DMA for _ in range(NBUF)],
            [pltpu.SemaphoreType.DMA for _ in range(NBUF)],
        ],
        compiler_params=pltpu.CompilerParams(use_tc_tiling_on_sc=False),
    )
    def k(prev_hbm, curr_hbm, embed_hbm, out_hbm,
          prev_v, curr_v, idx_vs, rows_vs, gsems, ssems):
        i32 = jnp.int32
        wid = lax.axis_index("s") * i32(NC) + lax.axis_index("c")
        base = wid * i32(b_per_w)

        pltpu.sync_copy(prev_hbm.at[pl.ds(base, b_per_w)], prev_v)
        pltpu.sync_copy(curr_hbm.at[pl.ds(base, b_per_w)], curr_v)

        def hash_chunk(ci, b):
            @pl.loop(i32(0), i32(NSTR))
            def hash_row(s):
                for g in range(IPS // LANES):
                    o = ci * i32(CHUNK) + s * i32(IPS) + i32(g * LANES)
                    p = prev_v[pl.ds(o, LANES)]
                    c = curr_v[pl.ds(o, LANES)]
                    a = (p * i32(16000)) % i32(BUCKETS)
                    b_ = (p * i32(15337)) % i32(BUCKETS)
                    idx_vs[b][s, pl.ds(g * LANES, LANES)] = (a + b_ + c) % i32(BUCKETS)

        def fire_gathers(b):
            return [pltpu.async_copy(
                embed_hbm.at[idx_vs[b].at[i32(s)]],
                rows_vs[b].at[pl.ds(s * IPS, IPS), :],
                gsems[b]) for s in range(NSTR)]

        def fire_store(ci, b):
            off = base + ci * i32(CHUNK)

            @pl.when(wid < i32(NW // 2))
            def _():
                pltpu.async_copy(
                    rows_vs[b],
                    out_hbm.at[pl.ds(off, CHUNK), pl.ds(i32(0), ED)],
                    ssems[b])

            @pl.when(wid >= i32(NW // 2))
            def _():
                pltpu.async_copy(
                    rows_vs[b],
                    out_hbm.at[pl.ds(off - i32(half), CHUNK), pl.ds(i32(ED), ED)],
                    ssems[b])

        def wait_store(b):
            pltpu.make_async_copy(
                rows_vs[b],
                out_hbm.at[pl.ds(i32(0), CHUNK), pl.ds(i32(0), ED)],
                ssems[b]).wait()

        @pl.loop(i32(0), i32(n_groups))
        def group_body(g):
            descs = []
            for b in range(NBUF):
                ci = g * i32(NBUF) + i32(b)

                @pl.when(g > i32(0))
                def _():
                    wait_store(b)

                hash_chunk(ci, b)
                descs.append(fire_gathers(b))
            for b in range(NBUF):
                ci = g * i32(NBUF) + i32(b)
                for d in descs[b]:
                    d.wait()
                fire_store(ci, b)

        for b in range(NBUF):
            wait_store(b)

    return k(prev, curr, embed)


def _tc_project(e2, W, total):
    half = total // 2
    PBLK = 2048

    def body(e_ref, w_ref, o_ref):
        x = e_ref[...]
        w = w_ref[...]
        lo = x[:, :ED]
        hi = x[:, ED:]
        o_ref[0] = lax.dot_general(
            lo, w, (((1,), (1,)), ((), ())), preferred_element_type=jnp.float32)
        o_ref[1] = lax.dot_general(
            hi, w, (((1,), (1,)), ((), ())), preferred_element_type=jnp.float32)

    return pl.pallas_call(
        body,
        grid=(half // PBLK,),
        in_specs=[
            pl.BlockSpec((PBLK, 2 * ED), lambda i: (i, jnp.int32(0))),
            pl.BlockSpec((MD, ED), lambda i: (jnp.int32(0), jnp.int32(0))),
        ],
        out_specs=pl.BlockSpec(
            (2, PBLK, MD), lambda i: (jnp.int32(0), i, jnp.int32(0))),
        out_shape=jax.ShapeDtypeStruct((2, half, MD), jnp.float32),
    )(e2, W)


def kernel(prev_tok, curr_tok, embed, W):
    B, L = prev_tok.shape
    total = B * L
    prev = prev_tok.astype(jnp.int32).reshape(total)
    curr = curr_tok.astype(jnp.int32).reshape(total)
    e2 = _sc_hash_gather(prev, curr, embed.astype(jnp.float32), total)
    out = _tc_project(e2, W.astype(jnp.float32), total)
    return out.reshape(B, L, MD)


# speedup vs baseline: 1.8350x; 1.4867x over previous
import functools

import jax
import jax.numpy as jnp
from jax import lax
from jax.experimental import pallas as pl
from jax.experimental.pallas import tpu as pltpu
from jax.experimental.pallas import tpu_sc as plsc

BUCKETS = 1000000
ED = 64
MD = 128
NC, NS, LANES = 2, 16, 16
NW = NC * NS

CHUNK = 512
IPS = 128
NSTR = CHUNK // IPS
NBUF = 2


def _sc_hash_gather(prev, curr, embed, total):
    half = total // 2
    b_per_w = total // NW
    n_chunks = b_per_w // CHUNK
    n_groups = n_chunks // NBUF

    @functools.partial(
        pl.kernel,
        out_type=jax.ShapeDtypeStruct((half, 2 * ED), jnp.float32),
        mesh=plsc.VectorSubcoreMesh(core_axis_name="c", subcore_axis_name="s"),
        scratch_types=[
            pltpu.VMEM((b_per_w,), jnp.int32),
            pltpu.VMEM((b_per_w,), jnp.int32),
            [pltpu.VMEM((NSTR, IPS), jnp.int32) for _ in range(NBUF)],
            [pltpu.VMEM((CHUNK, ED), jnp.float32) for _ in range(NBUF)],
            [pltpu.SemaphoreType.DMA for _ in range(NBUF)],
            [pltpu.SemaphoreType.DMA for _ in range(NBUF)],
        ],
        compiler_params=pltpu.CompilerParams(use_tc_tiling_on_sc=False),
    )
    def k(prev_hbm, curr_hbm, embed_hbm, out_hbm,
          prev_v, curr_v, idx_vs, rows_vs, gsems, ssems):
        i32 = jnp.int32
        wid = lax.axis_index("s") * i32(NC) + lax.axis_index("c")
        base = wid * i32(b_per_w)

        pltpu.sync_copy(prev_hbm.at[pl.ds(base, b_per_w)], prev_v)
        pltpu.sync_copy(curr_hbm.at[pl.ds(base, b_per_w)], curr_v)

        def hash_chunk(ci, b):
            @pl.loop(i32(0), i32(NSTR))
            def hash_row(s):
                for g in range(IPS // LANES):
                    o = ci * i32(CHUNK) + s * i32(IPS) + i32(g * LANES)
                    p = prev_v[pl.ds(o, LANES)]
                    c = curr_v[pl.ds(o, LANES)]
                    k1 = (p.astype(jnp.float32) * jnp.float32(1e-3)).astype(jnp.int32)
                    p2 = p - k1 * i32(1000)
                    sv = k1 * i32(337000) + p2 * i32(31337) + c
                    q = (sv.astype(jnp.float32) * jnp.float32(1e-6)).astype(jnp.int32)
                    r = sv - q * i32(BUCKETS)
                    r = jnp.where(r < i32(0), r + i32(BUCKETS), r)
                    r = jnp.where(r >= i32(BUCKETS), r - i32(BUCKETS), r)
                    idx_vs[b][s, pl.ds(g * LANES, LANES)] = r

        def fire_gathers(b):
            return [pltpu.async_copy(
                embed_hbm.at[idx_vs[b].at[i32(s)]],
                rows_vs[b].at[pl.ds(s * IPS, IPS), :],
                gsems[b]) for s in range(NSTR)]

        def fire_store(ci, b):
            off = base + ci * i32(CHUNK)

            @pl.when(wid < i32(NW // 2))
            def _():
                pltpu.async_copy(
                    rows_vs[b],
                    out_hbm.at[pl.ds(off, CHUNK), pl.ds(i32(0), ED)],
                    ssems[b])

            @pl.when(wid >= i32(NW // 2))
            def _():
                pltpu.async_copy(
                    rows_vs[b],
                    out_hbm.at[pl.ds(off - i32(half), CHUNK), pl.ds(i32(ED), ED)],
                    ssems[b])

        def wait_store(b):
            pltpu.make_async_copy(
                rows_vs[b],
                out_hbm.at[pl.ds(i32(0), CHUNK), pl.ds(i32(0), ED)],
                ssems[b]).wait()

        @pl.loop(i32(0), i32(n_groups))
        def group_body(g):
            descs = []
            for b in range(NBUF):
                ci = g * i32(NBUF) + i32(b)

                @pl.when(g > i32(0))
                def _():
                    wait_store(b)

                hash_chunk(ci, b)
                descs.append(fire_gathers(b))
            for b in range(NBUF):
                ci = g * i32(NBUF) + i32(b)
                for d in descs[b]:
                    d.wait()
                fire_store(ci, b)

        for b in range(NBUF):
            wait_store(b)

    return k(prev, curr, embed)


def _tc_project(e2, W, total):
    half = total // 2
    PBLK = 2048

    def body(e_ref, w_ref, o_ref):
        x = e_ref[...]
        w = w_ref[...]
        lo = x[:, :ED]
        hi = x[:, ED:]
        o_ref[0] = lax.dot_general(
            lo, w, (((1,), (1,)), ((), ())), preferred_element_type=jnp.float32)
        o_ref[1] = lax.dot_general(
            hi, w, (((1,), (1,)), ((), ())), preferred_element_type=jnp.float32)

    return pl.pallas_call(
        body,
        grid=(half // PBLK,),
        in_specs=[
            pl.BlockSpec((PBLK, 2 * ED), lambda i: (i, jnp.int32(0))),
            pl.BlockSpec((MD, ED), lambda i: (jnp.int32(0), jnp.int32(0))),
        ],
        out_specs=pl.BlockSpec(
            (2, PBLK, MD), lambda i: (jnp.int32(0), i, jnp.int32(0))),
        out_shape=jax.ShapeDtypeStruct((2, half, MD), jnp.float32),
    )(e2, W)


def kernel(prev_tok, curr_tok, embed, W):
    B, L = prev_tok.shape
    total = B * L
    prev = prev_tok.astype(jnp.int32).reshape(total)
    curr = curr_tok.astype(jnp.int32).reshape(total)
    e2 = _sc_hash_gather(prev, curr, embed.astype(jnp.float32), total)
    out = _tc_project(e2, W.astype(jnp.float32), total)
    return out.reshape(B, L, MD)


# speedup vs baseline: 2.7412x; 1.4938x over previous
import functools

import jax
import jax.numpy as jnp
from jax import lax
from jax.experimental import pallas as pl
from jax.experimental.pallas import tpu as pltpu
from jax.experimental.pallas import tpu_sc as plsc

BUCKETS = 1000000
ED = 64
MD = 128
NC, NS, LANES = 2, 16, 16
NW = NC * NS

CHUNK = 256
IPS = 128
NSTR = CHUNK // IPS
NBUF = 2
PBLK = 4096


def _tc_project_table(embedT, W):
    def body(x_ref, w_ref, o_ref):
        o_ref[...] = lax.dot_general(
            x_ref[...], w_ref[...],
            (((0,), (1,)), ((), ())),
            preferred_element_type=jnp.float32)

    grid = (BUCKETS + PBLK - 1) // PBLK
    return pl.pallas_call(
        body,
        grid=(grid,),
        in_specs=[
            pl.BlockSpec((ED, PBLK), lambda i: (jnp.int32(0), i)),
            pl.BlockSpec((MD, ED), lambda i: (jnp.int32(0), jnp.int32(0))),
        ],
        out_specs=pl.BlockSpec((PBLK, MD), lambda i: (i, jnp.int32(0))),
        out_shape=jax.ShapeDtypeStruct((BUCKETS, MD), jnp.float32),
    )(embedT, W)


def _sc_hash_gather(prev, curr, table, total):
    b_per_w = total // NW
    n_chunks = b_per_w // CHUNK
    n_groups = n_chunks // NBUF

    @functools.partial(
        pl.kernel,
        out_type=jax.ShapeDtypeStruct((total, MD), jnp.float32),
        mesh=plsc.VectorSubcoreMesh(core_axis_name="c", subcore_axis_name="s"),
        scratch_types=[
            pltpu.VMEM((b_per_w,), jnp.int32),
            pltpu.VMEM((b_per_w,), jnp.int32),
            [pltpu.VMEM((NSTR, IPS), jnp.int32) for _ in range(NBUF)],
            [pltpu.VMEM((CHUNK, MD), jnp.float32) for _ in range(NBUF)],
            [pltpu.SemaphoreType.DMA for _ in range(NBUF)],
            [pltpu.SemaphoreType.DMA for _ in range(NBUF)],
        ],
        compiler_params=pltpu.CompilerParams(use_tc_tiling_on_sc=False),
    )
    def k(prev_hbm, curr_hbm, table_hbm, out_hbm,
          prev_v, curr_v, idx_vs, rows_vs, gsems, ssems):
        i32 = jnp.int32
        wid = lax.axis_index("s") * i32(NC) + lax.axis_index("c")
        base = wid * i32(b_per_w)

        pltpu.sync_copy(prev_hbm.at[pl.ds(base, b_per_w)], prev_v)
        pltpu.sync_copy(curr_hbm.at[pl.ds(base, b_per_w)], curr_v)

        def hash_chunk(ci, b):
            @pl.loop(i32(0), i32(NSTR))
            def hash_row(s):
                for g in range(IPS // LANES):
                    o = ci * i32(CHUNK) + s * i32(IPS) + i32(g * LANES)
                    p = prev_v[pl.ds(o, LANES)]
                    c = curr_v[pl.ds(o, LANES)]
                    k1 = (p.astype(jnp.float32) * jnp.float32(1e-3)).astype(jnp.int32)
                    p2 = p - k1 * i32(1000)
                    sv = k1 * i32(337000) + p2 * i32(31337) + c
                    q = (sv.astype(jnp.float32) * jnp.float32(1e-6)).astype(jnp.int32)
                    r = sv - q * i32(BUCKETS)
                    r = jnp.where(r < i32(0), r + i32(BUCKETS), r)
                    r = jnp.where(r >= i32(BUCKETS), r - i32(BUCKETS), r)
                    idx_vs[b][s, pl.ds(g * LANES, LANES)] = r

        def fire_gathers(b):
            return [pltpu.async_copy(
                table_hbm.at[idx_vs[b].at[i32(s)]],
                rows_vs[b].at[pl.ds(s * IPS, IPS), :],
                gsems[b]) for s in range(NSTR)]

        def fire_store(ci, b):
            off = base + ci * i32(CHUNK)
            pltpu.async_copy(
                rows_vs[b], out_hbm.at[pl.ds(off, CHUNK), :], ssems[b])

        def wait_store(b):
            pltpu.make_async_copy(
                rows_vs[b], out_hbm.at[pl.ds(i32(0), CHUNK), :], ssems[b]).wait()

        @pl.loop(i32(0), i32(n_groups))
        def group_body(g):
            descs = []
            for b in range(NBUF):
                ci = g * i32(NBUF) + i32(b)

                @pl.when(g > i32(0))
                def _():
                    wait_store(b)

                hash_chunk(ci, b)
                descs.append(fire_gathers(b))
            for b in range(NBUF):
                ci = g * i32(NBUF) + i32(b)
                for d in descs[b]:
                    d.wait()
                fire_store(ci, b)

        for b in range(NBUF):
            wait_store(b)

    return k(prev, curr, table)


def kernel(prev_tok, curr_tok, embed, W):
    B, L = prev_tok.shape
    total = B * L
    prev = prev_tok.astype(jnp.int32).reshape(total)
    curr = curr_tok.astype(jnp.int32).reshape(total)
    P = _tc_project_table(embed.astype(jnp.float32).T, W.astype(jnp.float32))
    out = _sc_hash_gather(prev, curr, P, total)
    return out.reshape(B, L, MD)


# speedup vs baseline: 3.2189x; 1.1743x over previous
import functools

import jax
import jax.numpy as jnp
from jax import lax
from jax.experimental import pallas as pl
from jax.experimental.pallas import tpu as pltpu
from jax.experimental.pallas import tpu_sc as plsc

BUCKETS = 1000000
ED = 64
MD = 128
NC, NS, LANES = 2, 16, 16
NW = NC * NS

CHUNK = 256
IPS = 128
NSTR = CHUNK // IPS
NBUF = 2
PBLK = 16384


def _tc_project_table(embedT, W):
    def body(x_ref, w_ref, o_ref):
        o_ref[...] = lax.dot_general(
            x_ref[...], w_ref[...],
            (((0,), (1,)), ((), ())),
            preferred_element_type=jnp.float32)

    grid = (BUCKETS + PBLK - 1) // PBLK
    return pl.pallas_call(
        body,
        grid=(grid,),
        in_specs=[
            pl.BlockSpec((ED, PBLK), lambda i: (jnp.int32(0), i)),
            pl.BlockSpec((MD, ED), lambda i: (jnp.int32(0), jnp.int32(0))),
        ],
        out_specs=pl.BlockSpec((PBLK, MD), lambda i: (i, jnp.int32(0))),
        out_shape=jax.ShapeDtypeStruct((BUCKETS, MD), jnp.float32),
    )(embedT, W)


def _sc_hash_gather(prev, curr, table, total):
    b_per_w = total // NW
    n_chunks = b_per_w // CHUNK
    n_groups = n_chunks // NBUF

    @functools.partial(
        pl.kernel,
        out_type=jax.ShapeDtypeStruct((total, MD), jnp.float32),
        mesh=plsc.VectorSubcoreMesh(core_axis_name="c", subcore_axis_name="s"),
        scratch_types=[
            pltpu.VMEM((b_per_w,), jnp.int32),
            pltpu.VMEM((b_per_w,), jnp.int32),
            [pltpu.VMEM((NSTR, IPS), jnp.int32) for _ in range(NBUF)],
            [pltpu.VMEM((CHUNK, MD), jnp.float32) for _ in range(NBUF)],
            [pltpu.SemaphoreType.DMA for _ in range(NBUF)],
            [pltpu.SemaphoreType.DMA for _ in range(NBUF)],
        ],
        compiler_params=pltpu.CompilerParams(use_tc_tiling_on_sc=False),
    )
    def k(prev_hbm, curr_hbm, table_hbm, out_hbm,
          prev_v, curr_v, idx_vs, rows_vs, gsems, ssems):
        i32 = jnp.int32
        wid = lax.axis_index("s") * i32(NC) + lax.axis_index("c")
        base = wid * i32(b_per_w)

        pltpu.sync_copy(prev_hbm.at[pl.ds(base, b_per_w)], prev_v)
        pltpu.sync_copy(curr_hbm.at[pl.ds(base, b_per_w)], curr_v)

        def hash_chunk(ci, b):
            @pl.loop(i32(0), i32(NSTR))
            def hash_row(s):
                for g in range(IPS // LANES):
                    o = ci * i32(CHUNK) + s * i32(IPS) + i32(g * LANES)
                    p = prev_v[pl.ds(o, LANES)]
                    c = curr_v[pl.ds(o, LANES)]
                    k1 = (p.astype(jnp.float32) * jnp.float32(1e-3)).astype(jnp.int32)
                    p2 = p - k1 * i32(1000)
                    sv = k1 * i32(337000) + p2 * i32(31337) + c
                    q = (sv.astype(jnp.float32) * jnp.float32(1e-6)).astype(jnp.int32)
                    r = sv - q * i32(BUCKETS)
                    r = jnp.where(r < i32(0), r + i32(BUCKETS), r)
                    r = jnp.where(r >= i32(BUCKETS), r - i32(BUCKETS), r)
                    idx_vs[b][s, pl.ds(g * LANES, LANES)] = r

        def fire_gathers(b):
            return [pltpu.async_copy(
                table_hbm.at[idx_vs[b].at[i32(s)]],
                rows_vs[b].at[pl.ds(s * IPS, IPS), :],
                gsems[b]) for s in range(NSTR)]

        def fire_store(ci, b):
            off = base + ci * i32(CHUNK)
            pltpu.async_copy(
                rows_vs[b], out_hbm.at[pl.ds(off, CHUNK), :], ssems[b])

        def wait_store(b):
            pltpu.make_async_copy(
                rows_vs[b], out_hbm.at[pl.ds(i32(0), CHUNK), :], ssems[b]).wait()

        @pl.loop(i32(0), i32(n_groups))
        def group_body(g):
            descs = []
            for b in range(NBUF):
                ci = g * i32(NBUF) + i32(b)

                @pl.when(g > i32(0))
                def _():
                    wait_store(b)

                hash_chunk(ci, b)
                descs.append(fire_gathers(b))
            for b in range(NBUF):
                ci = g * i32(NBUF) + i32(b)
                for d in descs[b]:
                    d.wait()
                fire_store(ci, b)

        for b in range(NBUF):
            wait_store(b)

    return k(prev, curr, table)


def kernel(prev_tok, curr_tok, embed, W):
    B, L = prev_tok.shape
    total = B * L
    prev = prev_tok.astype(jnp.int32).reshape(total)
    curr = curr_tok.astype(jnp.int32).reshape(total)
    P = _tc_project_table(embed.astype(jnp.float32).T, W.astype(jnp.float32))
    out = _sc_hash_gather(prev, curr, P, total)
    return out.reshape(B, L, MD)


# speedup vs baseline: 3.2394x; 1.0064x over previous
import functools

import jax
import jax.numpy as jnp
from jax import lax
from jax.experimental import pallas as pl
from jax.experimental.pallas import tpu as pltpu
from jax.experimental.pallas import tpu_sc as plsc

BUCKETS = 1000000
ED = 64
MD = 128
NC, NS, LANES = 2, 16, 16
NW = NC * NS

CHUNK = 256
IPS = 128
NSTR = CHUNK // IPS
NBUF = 2
PBLK = 32768


def _tc_project_table(embedT, W):
    def body(x_ref, w_ref, o_ref):
        o_ref[...] = lax.dot_general(
            x_ref[...], w_ref[...],
            (((0,), (1,)), ((), ())),
            preferred_element_type=jnp.float32)

    grid = (BUCKETS + PBLK - 1) // PBLK
    return pl.pallas_call(
        body,
        grid=(grid,),
        in_specs=[
            pl.BlockSpec((ED, PBLK), lambda i: (jnp.int32(0), i)),
            pl.BlockSpec((MD, ED), lambda i: (jnp.int32(0), jnp.int32(0))),
        ],
        out_specs=pl.BlockSpec((PBLK, MD), lambda i: (i, jnp.int32(0))),
        out_shape=jax.ShapeDtypeStruct((BUCKETS, MD), jnp.float32),
    )(embedT, W)


def _sc_hash_gather(prev, curr, table, total):
    b_per_w = total // NW
    n_chunks = b_per_w // CHUNK
    n_groups = n_chunks // NBUF

    @functools.partial(
        pl.kernel,
        out_type=jax.ShapeDtypeStruct((total, MD), jnp.float32),
        mesh=plsc.VectorSubcoreMesh(core_axis_name="c", subcore_axis_name="s"),
        scratch_types=[
            pltpu.VMEM((b_per_w,), jnp.int32),
            pltpu.VMEM((b_per_w,), jnp.int32),
            [pltpu.VMEM((NSTR, IPS), jnp.int32) for _ in range(NBUF)],
            [pltpu.VMEM((CHUNK, MD), jnp.float32) for _ in range(NBUF)],
            [pltpu.SemaphoreType.DMA for _ in range(NBUF)],
            [pltpu.SemaphoreType.DMA for _ in range(NBUF)],
        ],
        compiler_params=pltpu.CompilerParams(use_tc_tiling_on_sc=False),
    )
    def k(prev_hbm, curr_hbm, table_hbm, out_hbm,
          prev_v, curr_v, idx_vs, rows_vs, gsems, ssems):
        i32 = jnp.int32
        wid = lax.axis_index("s") * i32(NC) + lax.axis_index("c")
        base = wid * i32(b_per_w)

        pltpu.sync_copy(prev_hbm.at[pl.ds(base, b_per_w)], prev_v)
        pltpu.sync_copy(curr_hbm.at[pl.ds(base, b_per_w)], curr_v)

        def hash_chunk(ci, b):
            @pl.loop(i32(0), i32(NSTR))
            def hash_row(s):
                for g in range(IPS // LANES):
                    o = ci * i32(CHUNK) + s * i32(IPS) + i32(g * LANES)
                    p = prev_v[pl.ds(o, LANES)]
                    c = curr_v[pl.ds(o, LANES)]
                    k1 = (p.astype(jnp.float32) * jnp.float32(1e-3)).astype(jnp.int32)
                    p2 = p - k1 * i32(1000)
                    sv = k1 * i32(337000) + p2 * i32(31337) + c
                    q = (sv.astype(jnp.float32) * jnp.float32(1e-6)).astype(jnp.int32)
                    r = sv - q * i32(BUCKETS)
                    r = jnp.where(r < i32(0), r + i32(BUCKETS), r)
                    r = jnp.where(r >= i32(BUCKETS), r - i32(BUCKETS), r)
                    idx_vs[b][s, pl.ds(g * LANES, LANES)] = r

        def fire_gathers(b):
            return [pltpu.async_copy(
                table_hbm.at[idx_vs[b].at[i32(s)]],
                rows_vs[b].at[pl.ds(s * IPS, IPS), :],
                gsems[b]) for s in range(NSTR)]

        def fire_store(ci, b):
            off = base + ci * i32(CHUNK)
            pltpu.async_copy(
                rows_vs[b], out_hbm.at[pl.ds(off, CHUNK), :], ssems[b])

        def wait_store(b):
            pltpu.make_async_copy(
                rows_vs[b], out_hbm.at[pl.ds(i32(0), CHUNK), :], ssems[b]).wait()

        @pl.loop(i32(0), i32(n_groups))
        def group_body(g):
            descs = []
            for b in range(NBUF):
                ci = g * i32(NBUF) + i32(b)

                @pl.when(g > i32(0))
                def _():
                    wait_store(b)

                hash_chunk(ci, b)
                descs.append(fire_gathers(b))
            for b in range(NBUF):
                ci = g * i32(NBUF) + i32(b)
                for d in descs[b]:
                    d.wait()
                fire_store(ci, b)

        for b in range(NBUF):
            wait_store(b)

    return k(prev, curr, table)


def kernel(prev_tok, curr_tok, embed, W):
    B, L = prev_tok.shape
    total = B * L
    prev = prev_tok.astype(jnp.int32).reshape(total)
    curr = curr_tok.astype(jnp.int32).reshape(total)
    P = _tc_project_table(embed.astype(jnp.float32).T, W.astype(jnp.float32))
    out = _sc_hash_gather(prev, curr, P, total)
    return out.reshape(B, L, MD)
